# baseline (speedup 1.0000x reference)
import jax
import jax.numpy as jnp
from jax import lax
from jax.experimental import pallas as pl
from jax.experimental.pallas import tpu as pltpu
from jax.experimental.pallas import tpu_sc as plsc

_N = 10000
_NP = 10048
_E = 160000
_EP = 163840
_D = 256
_DE = 16
_H = 8
_HH = 4
_F = 64
_HF = _H * _F
_FH = _HH * _F
_NEG = 0.2

_NC = 2
_NS = 16
_NT = _NP * _HH

_ET = _EP // _NS
_W = 1024
_NWIN = _ET // _W
_WR = _W // 128
_KB = 32
_CH = _NP // 4
_DUMP = 16
_ERH = _EP // 128


def _tc_node_body(x_ref, ws_ref, wd_ref, b_ref, u_ref, r_ref):
    x = x_ref[...]
    dn = (((1,), (1,)), ((), ()))
    u_ref[0] = lax.dot_general(x, ws_ref[0], dn,
                               preferred_element_type=jnp.float32)
    u_ref[1] = lax.dot_general(x, ws_ref[1], dn,
                               preferred_element_type=jnp.float32)
    r_ref[0] = lax.dot_general(x, wd_ref[0], dn,
                               preferred_element_type=jnp.float32) + b_ref[0]
    r_ref[1] = lax.dot_general(x, wd_ref[1], dn,
                               preferred_element_type=jnp.float32) + b_ref[1]


def _tc_attn_body(x_ref, was_ref, wad_ref, as_ref, ad_ref):
    x = x_ref[...]
    dn = (((1,), (1,)), ((), ()))
    as_ref[...] = lax.dot_general(x, was_ref[...], dn,
                                  preferred_element_type=jnp.float32)
    ad_ref[...] = lax.dot_general(x, wad_ref[...], dn,
                                  preferred_element_type=jnp.float32)


def _tc_edge_body(xe_ref, we_ref, ae_ref):
    a_e = lax.dot_general(we_ref[...], xe_ref[...],
                          (((1,), (1,)), ((), ())),
                          preferred_element_type=jnp.float32)
    ae_ref[...] = a_e.reshape(_H, -1, 128)


def _node_projections(feat_src, w_src, w_dst, b_dst):
    bn = 1256
    grid = _NP // bn
    ws = w_src.reshape(2, _FH, _D)
    wd = w_dst.reshape(2, _FH, _D)
    bb = b_dst.reshape(2, 1, _FH)
    return pl.pallas_call(
        _tc_node_body,
        grid=(grid,),
        in_specs=[
            pl.BlockSpec((bn, _D), lambda i: (i, 0)),
            pl.BlockSpec((2, _FH, _D), lambda i: (0, 0, 0)),
            pl.BlockSpec((2, _FH, _D), lambda i: (0, 0, 0)),
            pl.BlockSpec((2, 1, _FH), lambda i: (0, 0, 0)),
        ],
        out_specs=[
            pl.BlockSpec((2, bn, _FH), lambda i: (0, i, 0)),
            pl.BlockSpec((2, bn, _FH), lambda i: (0, i, 0)),
        ],
        out_shape=[
            jax.ShapeDtypeStruct((2, _NP, _FH), jnp.float32),
            jax.ShapeDtypeStruct((2, _NP, _FH), jnp.float32),
        ],
    )(feat_src, ws, wd, bb)


def _attn_projections(feat_src, w_as, w_ad):
    bn = 1256
    grid = _NP // bn
    return pl.pallas_call(
        _tc_attn_body,
        grid=(grid,),
        in_specs=[
            pl.BlockSpec((bn, _D), lambda i: (i, 0)),
            pl.BlockSpec((_H, _D), lambda i: (0, 0)),
            pl.BlockSpec((_H, _D), lambda i: (0, 0)),
        ],
        out_specs=[
            pl.BlockSpec((bn, _H), lambda i: (i, 0)),
            pl.BlockSpec((bn, _H), lambda i: (i, 0)),
        ],
        out_shape=[
            jax.ShapeDtypeStruct((_NP, _H), jnp.float32),
            jax.ShapeDtypeStruct((_NP, _H), jnp.float32),
        ],
    )(feat_src, w_as, w_ad)


def _edge_projection(feat_edge, w_ae):
    be = 8192
    grid = _EP // be
    return pl.pallas_call(
        _tc_edge_body,
        grid=(grid,),
        in_specs=[
            pl.BlockSpec((be, _DE), lambda i: (i, 0)),
            pl.BlockSpec((_H, _DE), lambda i: (0, 0)),
        ],
        out_specs=pl.BlockSpec((_H, be // 128, 128), lambda i: (0, i, 0)),
        out_shape=jax.ShapeDtypeStruct((_H, _ERH, 128), jnp.float32),
    )(feat_edge, w_ae)


def _sc_stats_body(src_h, dst_h, ae_h, as_h, ad_h,
                   ee_h, es_h,
                   srcw, dstw, aew, eewb, esidx, as_l, ad_l, zb, es_sp,
                   sem, sem2):
    cid = lax.axis_index("c")
    sid = lax.axis_index("s")

    pltpu.sync_copy(as_h.at[pl.ds(cid * _NT, _NT)], as_l)
    pltpu.sync_copy(ad_h.at[pl.ds(cid * _NT, _NT)], ad_l)

    def zbody(i, c):
        zb[pl.ds(i * 16, 16)] = jnp.zeros((16,), jnp.float32)
        return c
    lax.fori_loop(0, 2512 // 16, zbody, jnp.int32(0))
    pltpu.sync_copy(zb, es_sp.at[pl.ds(sid * 2512, 2512)])
    plsc.subcore_barrier()

    base_e = sid * _ET

    def win_body(wi, carry):
        woff = base_e + wi * _W
        rbase = sid * (_ET // 128) + wi * _WR
        pltpu.sync_copy(src_h.at[pl.ds(woff, _W)], srcw)
        pltpu.sync_copy(dst_h.at[pl.ds(woff, _W)], dstw)
        for h in range(_HH):
            pltpu.sync_copy(ae_h.at[cid * _HH + h, pl.ds(rbase, _WR), :],
                            aew.at[h])

        def vbody(v, cc):
            sv = srcw[pl.ds(v * 16, 16)]
            dv = dstw[pl.ds(v * 16, 16)]
            row = v // 8
            lane = (v % 8) * 16
            for h in range(_HH):
                av = plsc.load_gather(as_l, [sv * _HH + h])
                bv = plsc.load_gather(ad_l, [dv * _HH + h])
                ev = av + bv + aew[h, row, pl.ds(lane, 16)]
                ev = jnp.where(ev > 0, ev, _NEG * ev)
                eewb[h, row, pl.ds(lane, 16)] = jnp.exp(ev)
                esidx[h, row, pl.ds(lane, 16)] = dv * _HH + h
            return cc
        lax.fori_loop(0, _W // 16, vbody, jnp.int32(0))

        for h in range(_HH):
            pltpu.sync_copy(eewb.at[h],
                            ee_h.at[cid * _HH + h, pl.ds(rbase, _WR), :])
        cps = [pltpu.async_copy(eewb.at[h, k], es_sp.at[esidx.at[h, k]],
                                sem2, add=True)
               for h in range(_HH) for k in range(_WR)]
        for cp in cps:
            cp.wait()
        return carry
    lax.fori_loop(0, _NWIN, win_body, jnp.int32(0))
    plsc.subcore_barrier()

    pltpu.sync_copy(es_sp.at[pl.ds(sid * 2512, 2512)], zb)
    pltpu.sync_copy(zb, es_h.at[pl.ds(cid * _NT + sid * 2512, 2512)])


def _sc_stats(src, dst, aedge, as_s, ad_s):
    mesh = plsc.VectorSubcoreMesh(core_axis_name="c", subcore_axis_name="s")
    f32 = jnp.float32
    i32 = jnp.int32
    kern = pl.kernel(
        _sc_stats_body,
        out_type=(
            jax.ShapeDtypeStruct((_H, _ERH, 128), f32),
            jax.ShapeDtypeStruct((2 * _NT,), f32),
        ),
        mesh=mesh,
        scratch_types=[
            pltpu.VMEM((_W,), i32),
            pltpu.VMEM((_W,), i32),
            pltpu.VMEM((_HH, _WR, 128), f32),
            pltpu.VMEM((_HH, _WR, 128), f32),
            pltpu.VMEM((_HH, _WR, 128), i32),
            pltpu.VMEM((_NT,), f32),
            pltpu.VMEM((_NT,), f32),
            pltpu.VMEM((2512,), f32),
            pltpu.VMEM_SHARED((_NT,), f32),
            pltpu.SemaphoreType.DMA,
            pltpu.SemaphoreType.DMA,
        ],
        compiler_params=pltpu.CompilerParams(needs_layout_passes=False),
    )
    return kern(src, dst, aedge, as_s, ad_s)


def _sc_agg_body(src_h, dst_h, ee_h, es_h, u_h, r_h,
                 out_h,
                 srcw, dstw, eew, selb,
                 lvb0, dstb0, srcb0, ldstb0, wb0, ub0,
                 lvb1, dstb1, srcb1, ldstb1, wb1, ub1,
                 es_l, rbuf, acc, semg0, semg1):
    cid = lax.axis_index("c")
    sid = lax.axis_index("s")
    base_e = sid * _ET
    iota = lax.iota(jnp.int32, 16)

    pltpu.sync_copy(es_h.at[pl.ds(cid * _NT, _NT)], es_l.at[pl.ds(0, _NT)])
    es_l[pl.ds(_NT, 16)] = jnp.zeros((16,), jnp.float32) + 1.0

    for p in range(4):
        nbase = p * _CH

        @pl.when(sid < 15)
        def _():
            for k in range(4):
                loff = sid * 320 + k * 80
                pltpu.sync_copy(r_h.at[cid, pl.ds(2 * nbase + loff, 80), :],
                                rbuf)
                pltpu.sync_copy(rbuf, acc.at[pl.ds(loff, 80), :])

        @pl.when(sid == 15)
        def _():
            for k in range(2):
                loff = 4800 + k * 80
                pltpu.sync_copy(r_h.at[cid, pl.ds(2 * nbase + loff, 80), :],
                                rbuf)
                pltpu.sync_copy(rbuf, acc.at[pl.ds(loff, 80), :])
            pltpu.sync_copy(r_h.at[cid, pl.ds(2 * nbase + 4960, 64), :],
                            rbuf.at[pl.ds(0, 64), :])
            pltpu.sync_copy(rbuf.at[pl.ds(0, 64), :],
                            acc.at[pl.ds(4960, 64), :])
        plsc.subcore_barrier()

        def win_body(wi, carry):
            woff = base_e + wi * _W
            rbase = sid * (_ET // 128) + wi * _WR
            wcps = [
                pltpu.async_copy(src_h.at[pl.ds(woff, _W)],
                                 srcw.at[pl.ds(0, _W)], semg0),
                pltpu.async_copy(dst_h.at[pl.ds(woff, _W)],
                                 dstw.at[pl.ds(0, _W)], semg0),
            ]
            for h in range(_HH):
                wcps.append(pltpu.async_copy(
                    ee_h.at[cid * _HH + h, pl.ds(rbase, _WR), :],
                    eew.at[h], semg0))
            for cp in wcps:
                cp.wait()
            srcw[pl.ds(_W, 16)] = iota * 0
            dstw[pl.ds(_W, 16)] = iota * 0 + (nbase + _CH)

            def scan_body(v, off):
                dvec = dstw[pl.ds(v * 16, 16)]
                m = (dvec >= nbase) & (dvec < nbase + _CH)
                plsc.store_compressed(selb.at[pl.ds(off, 16)],
                                      v * 16 + iota, mask=m)
                return off + jnp.sum(m.astype(jnp.int32))
            n = lax.fori_loop(0, _W // 16, scan_body, jnp.int32(0))
            selb[pl.ds(n, 16)] = _W + iota
            selb[pl.ds(n + 16, 16)] = _W + iota
            nch = (n + _KB - 1) // _KB

            bufs = ((lvb0, dstb0, srcb0, ldstb0, wb0, ub0, semg0),
                    (lvb1, dstb1, srcb1, ldstb1, wb1, ub1, semg1))

            def build(c, bi):
                lvb, dstb, srcb, ldstb, wb, ub, semg = bufs[bi]

                @pl.when(c >= 2)
                def _():
                    pltpu.make_async_copy(ub, acc.at[ldstb], semg).wait()
                for q in range(_KB // 16):
                    lv = selb[pl.ds(c * _KB + q * 16, 16)]
                    sv = plsc.load_gather(srcw, [lv])
                    dv = plsc.load_gather(dstw, [lv])
                    lvb[pl.ds(q * 16, 16)] = lv
                    dstb[pl.ds(q * 16, 16)] = dv
                    rs = 2 * (sv + cid * _NP)
                    srcb[pl.ds(q * 16, 16)] = rs
                    srcb[pl.ds(_KB + q * 16, 16)] = rs + 1
                    ld = 2 * (dv - nbase)
                    ldstb[pl.ds(q * 16, 16)] = ld
                    ldstb[pl.ds(_KB + q * 16, 16)] = ld + 1
                pltpu.async_copy(u_h.at[srcb], ub, semg)

            def compute(c, bi):
                lvb, dstb, srcb, ldstb, wb, ub, semg = bufs[bi]
                for q in range(_KB // 16):
                    lv = lvb[pl.ds(q * 16, 16)]
                    dv = dstb[pl.ds(q * 16, 16)]
                    lvc = lv & (_W - 1)
                    for h in range(_HH):
                        eev = plsc.load_gather(
                            eew, [jnp.full((16,), h, jnp.int32),
                                  lvc >> 7, lvc & 127])
                        esv = plsc.load_gather(es_l, [dv * _HH + h])
                        wb[q * _HH + h, :] = eev / (esv + jnp.float32(1e-9))
                pltpu.make_async_copy(u_h.at[srcb], ub, semg).wait()

                def ebody(g, cc):
                    for k in range(8):
                        e = g * 8 + k
                        lanev = (e & 15).astype(jnp.int32)
                        wrowv = (e // 16) * _HH
                        for h in range(_HH):
                            wv = plsc.load_gather(
                                wb, [jnp.zeros((16,), jnp.int32) + wrowv + h,
                                     jnp.zeros((16,), jnp.int32) + lanev])
                            for t in range(4):
                                j = h * 4 + t
                                row = e + _KB * (j // 8)
                                s = pl.ds((j % 8) * 16, 16)
                                ub[row, s] = ub[row, s] * wv
                    return cc
                lax.fori_loop(0, _KB // 8, ebody, jnp.int32(0))
                pltpu.async_copy(ub, acc.at[ldstb], semg, add=True)

            @pl.when(nch > 0)
            def _():
                build(jnp.int32(0), 0)

            def pbody(c2, carry2):
                c0 = 2 * c2

                @pl.when(c0 + 1 < nch)
                def _():
                    build(c0 + 1, 1)

                @pl.when(c0 < nch)
                def _():
                    compute(c0, 0)

                @pl.when(c0 + 2 < nch)
                def _():
                    build(c0 + 2, 0)

                @pl.when(c0 + 1 < nch)
                def _():
                    compute(c0 + 1, 1)
                return carry2
            lax.fori_loop(0, (nch + 1) // 2, pbody, jnp.int32(0))

            @pl.when(nch >= 1)
            def _():
                pltpu.make_async_copy(ub0, acc.at[ldstb0], semg0).wait()

            @pl.when(nch >= 2)
            def _():
                pltpu.make_async_copy(ub1, acc.at[ldstb1], semg1).wait()
            return carry
        lax.fori_loop(0, _NWIN, win_body, jnp.int32(0))
        plsc.subcore_barrier()

        @pl.when(sid < 15)
        def _():
            for k in range(4):
                loff = sid * 320 + k * 80
                pltpu.sync_copy(acc.at[pl.ds(loff, 80), :], rbuf)
                pltpu.sync_copy(
                    rbuf, out_h.at[cid, pl.ds(2 * nbase + loff, 80), :])

        @pl.when(sid == 15)
        def _():
            for k in range(2):
                loff = 4800 + k * 80
                pltpu.sync_copy(acc.at[pl.ds(loff, 80), :], rbuf)
                pltpu.sync_copy(
                    rbuf, out_h.at[cid, pl.ds(2 * nbase + loff, 80), :])
            pltpu.sync_copy(acc.at[pl.ds(4960, 64), :],
                            rbuf.at[pl.ds(0, 64), :])
            pltpu.sync_copy(rbuf.at[pl.ds(0, 64), :],
                            out_h.at[cid, pl.ds(2 * nbase + 4960, 64), :])
        plsc.subcore_barrier()


def _sc_agg(src, dst, ee, es, u, r):
    mesh = plsc.VectorSubcoreMesh(core_axis_name="c", subcore_axis_name="s")
    f32 = jnp.float32
    i32 = jnp.int32
    kern = pl.kernel(
        _sc_agg_body,
        out_type=jax.ShapeDtypeStruct((2, 2 * _NP, 128), f32),
        mesh=mesh,
        scratch_types=[
            pltpu.VMEM((_W + 16,), i32),
            pltpu.VMEM((_W + 16,), i32),
            pltpu.VMEM((_HH, _WR, 128), f32),
            pltpu.VMEM((_W + 2 * _KB,), i32),
            pltpu.VMEM((_KB,), i32),
            pltpu.VMEM((_KB,), i32),
            pltpu.VMEM((2 * _KB,), i32),
            pltpu.VMEM((2 * _KB,), i32),
            pltpu.VMEM(((_KB // 16) * _HH, 16), f32),
            pltpu.VMEM((2 * _KB, 128), f32),
            pltpu.VMEM((_KB,), i32),
            pltpu.VMEM((_KB,), i32),
            pltpu.VMEM((2 * _KB,), i32),
            pltpu.VMEM((2 * _KB,), i32),
            pltpu.VMEM(((_KB // 16) * _HH, 16), f32),
            pltpu.VMEM((2 * _KB, 128), f32),
            pltpu.VMEM((_NT + 16,), f32),
            pltpu.VMEM((80, 128), f32),
            pltpu.VMEM_SHARED((2 * (_CH + _DUMP), 128), f32),
            pltpu.SemaphoreType.DMA,
            pltpu.SemaphoreType.DMA,
        ],
        compiler_params=pltpu.CompilerParams(needs_layout_passes=False),
    )
    return kern(src, dst, ee, es, u, r)


def kernel(feat_src, edge_index, feat_edge, W_src, W_dst, b_dst,
           W_attn_src, W_attn_dst, W_attn_edge):
    src = edge_index[0]
    dst = edge_index[1]
    src_p = jnp.pad(src, (0, _EP - _E))
    dst_p = jnp.pad(dst, (0, _EP - _E), constant_values=_NP - 1)
    feat_p = jnp.pad(feat_src, ((0, _NP - _N), (0, 0)))
    fe_p = jnp.pad(feat_edge, ((0, _EP - _E), (0, 0)))

    asrc, adst = _attn_projections(feat_p, W_attn_src, W_attn_dst)
    aedge = _edge_projection(fe_p, W_attn_edge)
    u, r = _node_projections(feat_p, W_src, W_dst, b_dst)

    as_s = jnp.concatenate([asrc[:, :_HH].reshape(-1),
                            asrc[:, _HH:].reshape(-1)])
    ad_s = jnp.concatenate([adst[:, :_HH].reshape(-1),
                            adst[:, _HH:].reshape(-1)])

    ee, es = _sc_stats(src_p, dst_p, aedge, as_s, ad_s)
    u2 = u.reshape(2 * _NP * 2, 128)
    r2 = r.reshape(2, 2 * _NP, 128)
    out = _sc_agg(src_p, dst_p, ee, es, u2, r2)
    out = out.reshape(2, _NP, _FH)
    rst = jnp.concatenate([out[0], out[1]], axis=1)
    return rst[:_N].reshape(_N, _H, _F)


# speedup vs baseline: 1.2108x; 1.2108x over previous
import jax
import jax.numpy as jnp
from jax import lax
from jax.experimental import pallas as pl
from jax.experimental.pallas import tpu as pltpu
from jax.experimental.pallas import tpu_sc as plsc

_N = 10000
_NP = 10048
_E = 160000
_EP = 163840
_D = 256
_DE = 16
_H = 8
_HH = 4
_F = 64
_HF = _H * _F
_FH = _HH * _F
_NEG = 0.2

_NC = 2
_NS = 16
_NT = _NP * _HH

_ET = _EP // _NS
_W = 2048
_NWIN = _ET // _W
_WR = _W // 128
_KB = 32
_CH = _NP // 4
_DUMP = 16
_ERH = _EP // 128


def _tc_node_body(x_ref, ws_ref, wd_ref, b_ref, u_ref, r_ref):
    x = x_ref[...]
    dn = (((1,), (1,)), ((), ()))
    u_ref[0] = lax.dot_general(x, ws_ref[0], dn,
                               preferred_element_type=jnp.float32)
    u_ref[1] = lax.dot_general(x, ws_ref[1], dn,
                               preferred_element_type=jnp.float32)
    r_ref[0] = lax.dot_general(x, wd_ref[0], dn,
                               preferred_element_type=jnp.float32) + b_ref[0]
    r_ref[1] = lax.dot_general(x, wd_ref[1], dn,
                               preferred_element_type=jnp.float32) + b_ref[1]


def _tc_attn_body(x_ref, was_ref, wad_ref, as_ref, ad_ref):
    x = x_ref[...]
    dn = (((1,), (1,)), ((), ()))
    as_ref[...] = lax.dot_general(x, was_ref[...], dn,
                                  preferred_element_type=jnp.float32)
    ad_ref[...] = lax.dot_general(x, wad_ref[...], dn,
                                  preferred_element_type=jnp.float32)


def _tc_edge_body(xe_ref, we_ref, ae_ref):
    a_e = lax.dot_general(we_ref[...], xe_ref[...],
                          (((1,), (1,)), ((), ())),
                          preferred_element_type=jnp.float32)
    ae_ref[...] = a_e.reshape(_H, -1, 128)


def _node_projections(feat_src, w_src, w_dst, b_dst):
    bn = 1256
    grid = _NP // bn
    ws = w_src.reshape(2, _FH, _D)
    wd = w_dst.reshape(2, _FH, _D)
    bb = b_dst.reshape(2, 1, _FH)
    return pl.pallas_call(
        _tc_node_body,
        grid=(grid,),
        in_specs=[
            pl.BlockSpec((bn, _D), lambda i: (i, 0)),
            pl.BlockSpec((2, _FH, _D), lambda i: (0, 0, 0)),
            pl.BlockSpec((2, _FH, _D), lambda i: (0, 0, 0)),
            pl.BlockSpec((2, 1, _FH), lambda i: (0, 0, 0)),
        ],
        out_specs=[
            pl.BlockSpec((2, bn, _FH), lambda i: (0, i, 0)),
            pl.BlockSpec((2, bn, _FH), lambda i: (0, i, 0)),
        ],
        out_shape=[
            jax.ShapeDtypeStruct((2, _NP, _FH), jnp.float32),
            jax.ShapeDtypeStruct((2, _NP, _FH), jnp.float32),
        ],
    )(feat_src, ws, wd, bb)


def _attn_projections(feat_src, w_as, w_ad):
    bn = 1256
    grid = _NP // bn
    return pl.pallas_call(
        _tc_attn_body,
        grid=(grid,),
        in_specs=[
            pl.BlockSpec((bn, _D), lambda i: (i, 0)),
            pl.BlockSpec((_H, _D), lambda i: (0, 0)),
            pl.BlockSpec((_H, _D), lambda i: (0, 0)),
        ],
        out_specs=[
            pl.BlockSpec((bn, _H), lambda i: (i, 0)),
            pl.BlockSpec((bn, _H), lambda i: (i, 0)),
        ],
        out_shape=[
            jax.ShapeDtypeStruct((_NP, _H), jnp.float32),
            jax.ShapeDtypeStruct((_NP, _H), jnp.float32),
        ],
    )(feat_src, w_as, w_ad)


def _edge_projection(feat_edge, w_ae):
    be = 8192
    grid = _EP // be
    return pl.pallas_call(
        _tc_edge_body,
        grid=(grid,),
        in_specs=[
            pl.BlockSpec((be, _DE), lambda i: (i, 0)),
            pl.BlockSpec((_H, _DE), lambda i: (0, 0)),
        ],
        out_specs=pl.BlockSpec((_H, be // 128, 128), lambda i: (0, i, 0)),
        out_shape=jax.ShapeDtypeStruct((_H, _ERH, 128), jnp.float32),
    )(feat_edge, w_ae)


def _sc_stats_body(src_h, dst_h, ae_h, as_h, ad_h,
                   ee_h, es_h,
                   srcw, dstw, aew, eewb, esidx, as_l, ad_l, zb, es_sp,
                   sem, sem2):
    cid = lax.axis_index("c")
    sid = lax.axis_index("s")

    pltpu.sync_copy(as_h.at[pl.ds(cid * _NT, _NT)], as_l)
    pltpu.sync_copy(ad_h.at[pl.ds(cid * _NT, _NT)], ad_l)

    def zbody(i, c):
        zb[pl.ds(i * 16, 16)] = jnp.zeros((16,), jnp.float32)
        return c
    lax.fori_loop(0, 2512 // 16, zbody, jnp.int32(0))
    pltpu.sync_copy(zb, es_sp.at[pl.ds(sid * 2512, 2512)])
    plsc.subcore_barrier()

    base_e = sid * _ET

    def win_body(wi, carry):
        woff = base_e + wi * _W
        rbase = sid * (_ET // 128) + wi * _WR
        pltpu.sync_copy(src_h.at[pl.ds(woff, _W)], srcw)
        pltpu.sync_copy(dst_h.at[pl.ds(woff, _W)], dstw)
        for h in range(_HH):
            pltpu.sync_copy(ae_h.at[cid * _HH + h, pl.ds(rbase, _WR), :],
                            aew.at[h])

        def vbody(v, cc):
            sv = srcw[pl.ds(v * 16, 16)]
            dv = dstw[pl.ds(v * 16, 16)]
            row = v // 8
            lane = (v % 8) * 16
            for h in range(_HH):
                av = plsc.load_gather(as_l, [sv * _HH + h])
                bv = plsc.load_gather(ad_l, [dv * _HH + h])
                ev = av + bv + aew[h, row, pl.ds(lane, 16)]
                ev = jnp.where(ev > 0, ev, _NEG * ev)
                eewb[h, row, pl.ds(lane, 16)] = jnp.exp(ev)
                esidx[h, row, pl.ds(lane, 16)] = dv * _HH + h
            return cc
        lax.fori_loop(0, _W // 16, vbody, jnp.int32(0))

        for h in range(_HH):
            pltpu.sync_copy(eewb.at[h],
                            ee_h.at[cid * _HH + h, pl.ds(rbase, _WR), :])
        cps = [pltpu.async_copy(eewb.at[h, k], es_sp.at[esidx.at[h, k]],
                                sem2, add=True)
               for h in range(_HH) for k in range(_WR)]
        for cp in cps:
            cp.wait()
        return carry
    lax.fori_loop(0, _NWIN, win_body, jnp.int32(0))
    plsc.subcore_barrier()

    pltpu.sync_copy(es_sp.at[pl.ds(sid * 2512, 2512)], zb)
    pltpu.sync_copy(zb, es_h.at[pl.ds(cid * _NT + sid * 2512, 2512)])


def _sc_stats(src, dst, aedge, as_s, ad_s):
    mesh = plsc.VectorSubcoreMesh(core_axis_name="c", subcore_axis_name="s")
    f32 = jnp.float32
    i32 = jnp.int32
    kern = pl.kernel(
        _sc_stats_body,
        out_type=(
            jax.ShapeDtypeStruct((_H, _ERH, 128), f32),
            jax.ShapeDtypeStruct((2 * _NT,), f32),
        ),
        mesh=mesh,
        scratch_types=[
            pltpu.VMEM((_W,), i32),
            pltpu.VMEM((_W,), i32),
            pltpu.VMEM((_HH, _WR, 128), f32),
            pltpu.VMEM((_HH, _WR, 128), f32),
            pltpu.VMEM((_HH, _WR, 128), i32),
            pltpu.VMEM((_NT,), f32),
            pltpu.VMEM((_NT,), f32),
            pltpu.VMEM((2512,), f32),
            pltpu.VMEM_SHARED((_NT,), f32),
            pltpu.SemaphoreType.DMA,
            pltpu.SemaphoreType.DMA,
        ],
        compiler_params=pltpu.CompilerParams(needs_layout_passes=False),
    )
    return kern(src, dst, aedge, as_s, ad_s)


def _sc_agg_body(src_h, dst_h, ee_h, es_h, u_h, r_h,
                 out_h,
                 srcw, dstw, eew, selb,
                 lvb0, dstb0, srcb0, ldstb0, wb0, ub0,
                 lvb1, dstb1, srcb1, ldstb1, wb1, ub1,
                 es_l, rbuf, acc, semg0, semg1):
    cid = lax.axis_index("c")
    sid = lax.axis_index("s")
    base_e = sid * _ET
    iota = lax.iota(jnp.int32, 16)

    pltpu.sync_copy(es_h.at[pl.ds(cid * _NT, _NT)], es_l.at[pl.ds(0, _NT)])
    es_l[pl.ds(_NT, 16)] = jnp.zeros((16,), jnp.float32) + 1.0

    for p in range(4):
        nbase = p * _CH

        @pl.when(sid < 15)
        def _():
            for k in range(4):
                loff = sid * 320 + k * 80
                pltpu.sync_copy(r_h.at[cid, pl.ds(2 * nbase + loff, 80), :],
                                rbuf)
                pltpu.sync_copy(rbuf, acc.at[pl.ds(loff, 80), :])

        @pl.when(sid == 15)
        def _():
            for k in range(2):
                loff = 4800 + k * 80
                pltpu.sync_copy(r_h.at[cid, pl.ds(2 * nbase + loff, 80), :],
                                rbuf)
                pltpu.sync_copy(rbuf, acc.at[pl.ds(loff, 80), :])
            pltpu.sync_copy(r_h.at[cid, pl.ds(2 * nbase + 4960, 64), :],
                            rbuf.at[pl.ds(0, 64), :])
            pltpu.sync_copy(rbuf.at[pl.ds(0, 64), :],
                            acc.at[pl.ds(4960, 64), :])
        plsc.subcore_barrier()

        def win_body(wi, carry):
            woff = base_e + wi * _W
            rbase = sid * (_ET // 128) + wi * _WR
            wcps = [
                pltpu.async_copy(src_h.at[pl.ds(woff, _W)],
                                 srcw.at[pl.ds(0, _W)], semg0),
                pltpu.async_copy(dst_h.at[pl.ds(woff, _W)],
                                 dstw.at[pl.ds(0, _W)], semg0),
            ]
            for h in range(_HH):
                wcps.append(pltpu.async_copy(
                    ee_h.at[cid * _HH + h, pl.ds(rbase, _WR), :],
                    eew.at[h], semg0))
            for cp in wcps:
                cp.wait()
            srcw[pl.ds(_W, 16)] = iota * 0
            dstw[pl.ds(_W, 16)] = iota * 0 + (nbase + _CH)

            def scan_body(v, off):
                dvec = dstw[pl.ds(v * 16, 16)]
                m = (dvec >= nbase) & (dvec < nbase + _CH)
                plsc.store_compressed(selb.at[pl.ds(off, 16)],
                                      v * 16 + iota, mask=m)
                return off + jnp.sum(m.astype(jnp.int32))
            n = lax.fori_loop(0, _W // 16, scan_body, jnp.int32(0))
            selb[pl.ds(n, 16)] = _W + iota
            selb[pl.ds(n + 16, 16)] = _W + iota
            nch = (n + _KB - 1) // _KB

            bufs = ((lvb0, dstb0, srcb0, ldstb0, wb0, ub0, semg0),
                    (lvb1, dstb1, srcb1, ldstb1, wb1, ub1, semg1))

            def build(c, bi):
                lvb, dstb, srcb, ldstb, wb, ub, semg = bufs[bi]

                @pl.when(c >= 2)
                def _():
                    pltpu.make_async_copy(ub, acc.at[ldstb], semg).wait()
                for q in range(_KB // 16):
                    lv = selb[pl.ds(c * _KB + q * 16, 16)]
                    sv = plsc.load_gather(srcw, [lv])
                    dv = plsc.load_gather(dstw, [lv])
                    lvb[pl.ds(q * 16, 16)] = lv
                    dstb[pl.ds(q * 16, 16)] = dv
                    rs = 2 * (sv + cid * _NP)
                    srcb[pl.ds(q * 16, 16)] = rs
                    srcb[pl.ds(_KB + q * 16, 16)] = rs + 1
                    ld = 2 * (dv - nbase)
                    ldstb[pl.ds(q * 16, 16)] = ld
                    ldstb[pl.ds(_KB + q * 16, 16)] = ld + 1
                pltpu.async_copy(u_h.at[srcb], ub, semg)

            def compute(c, bi):
                lvb, dstb, srcb, ldstb, wb, ub, semg = bufs[bi]
                for q in range(_KB // 16):
                    lv = lvb[pl.ds(q * 16, 16)]
                    dv = dstb[pl.ds(q * 16, 16)]
                    lvc = lv & (_W - 1)
                    for h in range(_HH):
                        eev = plsc.load_gather(
                            eew, [jnp.full((16,), h, jnp.int32),
                                  lvc >> 7, lvc & 127])
                        esv = plsc.load_gather(es_l, [dv * _HH + h])
                        wb[q * _HH + h, :] = eev / (esv + jnp.float32(1e-9))
                pltpu.make_async_copy(u_h.at[srcb], ub, semg).wait()

                @plsc.parallel_loop(0, _KB, step=1, unroll=4)
                def _(e):
                    lane = e & 15
                    wrow = (e // 16) * _HH
                    for h in range(_HH):
                        wv = plsc.load_gather(
                            wb, [jnp.full((16,), wrow + h, jnp.int32),
                                 jnp.full((16,), lane, jnp.int32)])
                        for t in range(4):
                            j = h * 4 + t
                            row = e + _KB * (j // 8)
                            s = pl.ds((j % 8) * 16, 16)
                            ub[row, s] = ub[row, s] * wv
                pltpu.async_copy(ub, acc.at[ldstb], semg, add=True)

            @pl.when(nch > 0)
            def _():
                build(jnp.int32(0), 0)

            def pbody(c2, carry2):
                c0 = 2 * c2

                @pl.when(c0 + 1 < nch)
                def _():
                    build(c0 + 1, 1)

                @pl.when(c0 < nch)
                def _():
                    compute(c0, 0)

                @pl.when(c0 + 2 < nch)
                def _():
                    build(c0 + 2, 0)

                @pl.when(c0 + 1 < nch)
                def _():
                    compute(c0 + 1, 1)
                return carry2
            lax.fori_loop(0, (nch + 1) // 2, pbody, jnp.int32(0))

            @pl.when(nch >= 1)
            def _():
                pltpu.make_async_copy(ub0, acc.at[ldstb0], semg0).wait()

            @pl.when(nch >= 2)
            def _():
                pltpu.make_async_copy(ub1, acc.at[ldstb1], semg1).wait()
            return carry
        lax.fori_loop(0, _NWIN, win_body, jnp.int32(0))
        plsc.subcore_barrier()

        @pl.when(sid < 15)
        def _():
            for k in range(4):
                loff = sid * 320 + k * 80
                pltpu.sync_copy(acc.at[pl.ds(loff, 80), :], rbuf)
                pltpu.sync_copy(
                    rbuf, out_h.at[cid, pl.ds(2 * nbase + loff, 80), :])

        @pl.when(sid == 15)
        def _():
            for k in range(2):
                loff = 4800 + k * 80
                pltpu.sync_copy(acc.at[pl.ds(loff, 80), :], rbuf)
                pltpu.sync_copy(
                    rbuf, out_h.at[cid, pl.ds(2 * nbase + loff, 80), :])
            pltpu.sync_copy(acc.at[pl.ds(4960, 64), :],
                            rbuf.at[pl.ds(0, 64), :])
            pltpu.sync_copy(rbuf.at[pl.ds(0, 64), :],
                            out_h.at[cid, pl.ds(2 * nbase + 4960, 64), :])
        plsc.subcore_barrier()


def _sc_agg(src, dst, ee, es, u, r):
    mesh = plsc.VectorSubcoreMesh(core_axis_name="c", subcore_axis_name="s")
    f32 = jnp.float32
    i32 = jnp.int32
    kern = pl.kernel(
        _sc_agg_body,
        out_type=jax.ShapeDtypeStruct((2, 2 * _NP, 128), f32),
        mesh=mesh,
        scratch_types=[
            pltpu.VMEM((_W + 16,), i32),
            pltpu.VMEM((_W + 16,), i32),
            pltpu.VMEM((_HH, _WR, 128), f32),
            pltpu.VMEM((_W + 2 * _KB,), i32),
            pltpu.VMEM((_KB,), i32),
            pltpu.VMEM((_KB,), i32),
            pltpu.VMEM((2 * _KB,), i32),
            pltpu.VMEM((2 * _KB,), i32),
            pltpu.VMEM(((_KB // 16) * _HH, 16), f32),
            pltpu.VMEM((2 * _KB, 128), f32),
            pltpu.VMEM((_KB,), i32),
            pltpu.VMEM((_KB,), i32),
            pltpu.VMEM((2 * _KB,), i32),
            pltpu.VMEM((2 * _KB,), i32),
            pltpu.VMEM(((_KB // 16) * _HH, 16), f32),
            pltpu.VMEM((2 * _KB, 128), f32),
            pltpu.VMEM((_NT + 16,), f32),
            pltpu.VMEM((80, 128), f32),
            pltpu.VMEM_SHARED((2 * (_CH + _DUMP), 128), f32),
            pltpu.SemaphoreType.DMA,
            pltpu.SemaphoreType.DMA,
        ],
        compiler_params=pltpu.CompilerParams(needs_layout_passes=False),
    )
    return kern(src, dst, ee, es, u, r)


def kernel(feat_src, edge_index, feat_edge, W_src, W_dst, b_dst,
           W_attn_src, W_attn_dst, W_attn_edge):
    src = edge_index[0]
    dst = edge_index[1]
    src_p = jnp.pad(src, (0, _EP - _E))
    dst_p = jnp.pad(dst, (0, _EP - _E), constant_values=_NP - 1)
    feat_p = jnp.pad(feat_src, ((0, _NP - _N), (0, 0)))
    fe_p = jnp.pad(feat_edge, ((0, _EP - _E), (0, 0)))

    asrc, adst = _attn_projections(feat_p, W_attn_src, W_attn_dst)
    aedge = _edge_projection(fe_p, W_attn_edge)
    u, r = _node_projections(feat_p, W_src, W_dst, b_dst)

    as_s = jnp.concatenate([asrc[:, :_HH].reshape(-1),
                            asrc[:, _HH:].reshape(-1)])
    ad_s = jnp.concatenate([adst[:, :_HH].reshape(-1),
                            adst[:, _HH:].reshape(-1)])

    ee, es = _sc_stats(src_p, dst_p, aedge, as_s, ad_s)
    u2 = u.reshape(2 * _NP * 2, 128)
    r2 = r.reshape(2, 2 * _NP, 128)
    out = _sc_agg(src_p, dst_p, ee, es, u2, r2)
    out = out.reshape(2, _NP, _FH)
    rst = jnp.concatenate([out[0], out[1]], axis=1)
    return rst[:_N].reshape(_N, _H, _F)
